# baseline (speedup 1.0000x reference)
import functools
import math

import jax
import jax.numpy as jnp
import numpy as np
from jax import lax
from jax.experimental import pallas as pl
from jax.experimental.pallas import tpu as pltpu
from jax.experimental.pallas import tpu_sc as plsc

Q = 1024
K = 100000
KPAD = 102400
D = 128
BK = 2048

NC = 2
NT = 16
L = 16
NWORK = NC * NT
ROWS_PER_TILE = Q // NWORK
W = 5
WE = KPAD // W
WL = WE // L
NB = WE // 128
NBINS = 256 * W * L
QK = Q * K
NOUT = QK + (KPAD - K)
MININT = np.int32(-(2 ** 31))


def _sims_body(q_ref, g_ref, out_ref):
    logits = jax.lax.dot_general(
        q_ref[...], g_ref[...],
        dimension_numbers=(((1,), (1,)), ((), ())),
        preferred_element_type=jnp.float32,
    ) / jnp.sqrt(jnp.asarray(D, jnp.float32))
    out_ref[...] = jax.nn.sigmoid(logits)


def _sims(q_features, g_features):
    g_pad = jnp.pad(g_features, ((0, KPAD - K), (0, 0)))
    grid = KPAD // BK
    return pl.pallas_call(
        _sims_body,
        grid=(grid,),
        in_specs=[
            pl.BlockSpec((Q, D), lambda i: (0, 0)),
            pl.BlockSpec((BK, D), lambda i: (i, 0)),
        ],
        out_specs=pl.BlockSpec((Q, BK), lambda i: (0, i)),
        out_shape=jax.ShapeDtypeStruct((Q, KPAD), jnp.float32),
    )(q_features, g_pad)


def _make_pass_body(pass_no):
    shift = 8 * pass_no
    first = pass_no == 0
    last = pass_no == 3

    def body(ink, inp, outk, outp, kwin, pwin, hist, kdata, pdata, destb,
             semk, semp):
        c = lax.axis_index("c")
        tid = lax.axis_index("s")
        wid = c * NT + tid
        lane = lax.iota(jnp.int32, L)
        zeros16 = jnp.zeros((L,), jnp.int32)
        ones16 = jnp.ones((L,), jnp.int32)

        def load_kp(w, s):
            pos = lane * WL + s
            if first:
                kf = plsc.load_gather(kwin, [pos])
                b = plsc.bitcast(kf, jnp.int32)
                u = jnp.bitwise_not(jnp.bitwise_xor(b, zeros16 + MININT))
                gidx = pos + w * WE
                key = jnp.where(gidx < zeros16 + K, u, zeros16 - 1)
                payload = gidx
            else:
                key = plsc.load_gather(kwin, [pos])
                payload = plsc.load_gather(pwin, [pos])
            return key, payload

        def stage(w, r, need_p):
            src = pl.multiple_of(r * KPAD + w * WE, 256)
            pltpu.sync_copy(ink.at[pl.ds(src, WE)], kwin)
            if need_p:
                pltpu.sync_copy(inp.at[pl.ds(src, WE)], pwin)

        def row_body(rl, _):
            r = wid * ROWS_PER_TILE + rl

            def zb(i, _):
                hist[pl.ds(pl.multiple_of(i * L, 16), L)] = zeros16
                return 0
            lax.fori_loop(0, NBINS // L, zb, 0)

            for w in range(W):
                stage(w, r, need_p=False)

                def ha(s, _):
                    key, _p = load_kp(w, s)
                    digit = (key >> (zeros16 + shift)) & (zeros16 + 255)
                    flat = digit * (W * L) + (w * L) + lane
                    plsc.addupdate_scatter(hist, [flat], ones16)
                    return 0
                lax.fori_loop(0, WL, ha, 0)

            def sb(i, carry):
                off = pl.multiple_of(i * L, 16)
                v = hist[pl.ds(off, L)]
                incl = plsc.cumsum(v)
                hist[pl.ds(off, L)] = incl - v + carry
                return carry + jnp.sum(v)
            lax.fori_loop(0, NBINS // L, sb, jnp.int32(0))

            for w in range(W):
                stage(w, r, need_p=not first)

                def mk(par):
                    return (
                        pltpu.make_async_copy(kdata.at[par],
                                              outk.at[destb.at[par]], semk),
                        pltpu.make_async_copy(pdata.at[par],
                                              outp.at[destb.at[par]], semp),
                    )

                def gb(bi, _):
                    par = bi & 1

                    @pl.when(bi >= 2)
                    def _():
                        for cp in mk(par):
                            cp.wait()
                    for j in range(8):
                        s = bi * 8 + j
                        key, payload = load_kp(w, s)
                        digit = (key >> (zeros16 + shift)) & (zeros16 + 255)
                        flat = digit * (W * L) + (w * L) + lane
                        dest = plsc.load_gather(hist, [flat])
                        plsc.addupdate_scatter(hist, [flat], ones16)
                        if last:
                            is_pad = digit > zeros16 + 127
                            dest = jnp.where(is_pad, dest + (QK - K),
                                             dest + r * K)
                            out_val = plsc.bitcast(jnp.bitwise_not(key),
                                                   jnp.float32)
                        else:
                            dest = dest + r * KPAD
                            out_val = key
                        jo = pl.ds(j * L, L)
                        kdata[par, jo] = out_val
                        pdata[par, jo] = payload
                        destb[par, jo] = dest
                    for cp in mk(par):
                        cp.start()
                    return 0
                lax.fori_loop(0, NB, gb, 0)

                for par in (0, 1):
                    for cp in mk(par):
                        cp.wait()
            return 0

        lax.fori_loop(0, ROWS_PER_TILE, row_body, 0)

    return body


def _make_pass(pass_no, in_types, out_types):
    mesh = plsc.VectorSubcoreMesh(core_axis_name="c", subcore_axis_name="s")
    kdtype = jnp.float32 if pass_no == 3 else jnp.int32
    kwin_dtype = jnp.float32 if pass_no == 0 else jnp.int32
    return pl.kernel(
        _make_pass_body(pass_no), mesh=mesh,
        compiler_params=pltpu.CompilerParams(needs_layout_passes=False),
        out_type=out_types,
        scratch_types=[
            pltpu.VMEM((WE,), kwin_dtype),
            pltpu.VMEM((WE,), jnp.int32),
            pltpu.VMEM((NBINS,), jnp.int32),
            pltpu.VMEM((2, 128), kdtype),
            pltpu.VMEM((2, 128), jnp.int32),
            pltpu.VMEM((2, 128), jnp.int32),
            pltpu.SemaphoreType.DMA,
            pltpu.SemaphoreType.DMA,
        ],
    )


def _sort(sims_flat):
    ii = jax.ShapeDtypeStruct((Q * KPAD,), jnp.int32)
    p0 = _make_pass(0, None, (ii, ii))
    p1 = _make_pass(1, None, (ii, ii))
    p2 = _make_pass(2, None, (ii, ii))
    p3 = _make_pass(3, None, (jax.ShapeDtypeStruct((NOUT,), jnp.float32),
                              jax.ShapeDtypeStruct((NOUT,), jnp.int32)))
    ak, ap = p0(sims_flat, sims_flat)
    bk, bp = p1(ak, ap)
    ak, ap = p2(bk, bp)
    vals, ranks = p3(ak, ap)
    return vals, ranks


def kernel(q_features, g_features):
    sims = _sims(q_features, g_features)
    vals, ranks = _sort(sims.reshape(Q * KPAD))
    return (vals[:QK].reshape(Q, K), ranks[:QK].reshape(Q, K))
